# baseline (speedup 1.0000x reference)
import functools
import jax
import jax.numpy as jnp
from jax import lax
from jax.experimental import pallas as pl
from jax.experimental.pallas import tpu as pltpu
from jax.experimental.pallas import tpu_sc as plsc

N_CAT = 26
N_NUM = 13
K_PAD = 32
N_TOK = 1 + N_CAT + N_NUM
G = 8
L = 16


def _sc_info():
    try:
        info = plsc.get_sparse_core_info()
        return info.num_cores, info.num_subcores
    except Exception:
        return 2, 16


@functools.partial(jax.jit, static_argnums=(6, 7))
def _tokenize(idx_pad, num_rep, emb_flat, w_flat, b_flat, cls_flat, B, D):
    NC, NS = _sc_info()
    NW = NC * NS
    assert B % (NW * G) == 0
    b_per_w = B // NW
    n_groups = b_per_w // G
    KD16 = D // L

    mesh = plsc.VectorSubcoreMesh(core_axis_name="c", subcore_axis_name="s")

    @functools.partial(
        pl.kernel,
        out_type=jax.ShapeDtypeStruct((B * N_TOK, D), jnp.float32),
        mesh=mesh,
        scratch_types=[
            pltpu.VMEM((b_per_w * K_PAD,), jnp.int32),
            pltpu.VMEM((b_per_w * N_NUM * L,), jnp.float32),
            pltpu.VMEM((N_NUM * D,), jnp.float32),
            pltpu.VMEM((N_NUM * D,), jnp.float32),
            pltpu.VMEM((D,), jnp.float32),
            pltpu.VMEM((G * N_TOK, D), jnp.float32),
            pltpu.SemaphoreType.DMA,
        ],
        compiler_params=pltpu.CompilerParams(use_tc_tiling_on_sc=False),
    )
    def k(idx_hbm, num_hbm, emb_hbm, w_hbm, bias_hbm, cls_hbm, out_hbm,
          idx_v, num_v, w_v, bias_v, cls_v, buf, sem_g):
        c = lax.axis_index("c")
        s = lax.axis_index("s")
        wid = s * NC + c
        base = wid * b_per_w

        pltpu.sync_copy(idx_hbm.at[pl.ds(base * K_PAD, b_per_w * K_PAD)], idx_v)
        pltpu.sync_copy(
            num_hbm.at[pl.ds(base * N_NUM * L, b_per_w * N_NUM * L)], num_v)
        pltpu.sync_copy(w_hbm, w_v)
        pltpu.sync_copy(bias_hbm, bias_v)
        pltpu.sync_copy(cls_hbm, cls_v)

        def group(g, carry):
            row0 = g * G
            copies = []
            for r in range(G):
                lr = row0 + r
                copies.append(pltpu.async_copy(
                    emb_hbm.at[idx_v.at[pl.ds(lr * K_PAD, N_CAT)]],
                    buf.at[pl.ds(r * N_TOK + 1, N_CAT)],
                    sem_g,
                ))
            for r in range(G):
                lr = row0 + r
                o = r * N_TOK
                for kk in range(KD16):
                    buf[o, pl.ds(kk * L, L)] = cls_v[pl.ds(kk * L, L)]
                for j in range(N_NUM):
                    nb = num_v[pl.ds((lr * N_NUM + j) * L, L)]
                    for kk in range(KD16):
                        wv = w_v[pl.ds((j * D) + kk * L, L)]
                        bv = bias_v[pl.ds((j * D) + kk * L, L)]
                        buf[o + 1 + N_CAT + j, pl.ds(kk * L, L)] = nb * wv + bv
            for cp in copies:
                cp.wait()
            pltpu.sync_copy(
                buf, out_hbm.at[pl.ds((base + row0) * N_TOK, G * N_TOK)])
            return carry

        lax.fori_loop(0, n_groups, group, 0)

    return k(idx_pad, num_rep, emb_flat, w_flat, b_flat, cls_flat)


def kernel(cat, num, emb_cat, w_num, b_num, cls):
    B = cat.shape[0]
    n_cat, vocab, D = emb_cat.shape
    offs = (jnp.arange(n_cat, dtype=jnp.int32) * vocab)[None, :]
    idx = cat.astype(jnp.int32) + offs
    idx_pad = jnp.zeros((B, K_PAD), jnp.int32).at[:, :n_cat].set(idx).reshape(-1)
    num_rep = jnp.broadcast_to(num[:, :, None], (B, num.shape[1], L)).reshape(-1)
    out = _tokenize(
        idx_pad,
        num_rep,
        emb_cat.reshape(n_cat * vocab, D),
        w_num.reshape(-1),
        b_num.reshape(-1),
        cls.reshape(-1),
        B, D,
    )
    return out.reshape(B, N_TOK, D)


# speedup vs baseline: 1.0081x; 1.0081x over previous
import functools
import jax
import jax.numpy as jnp
from jax import lax
from jax.experimental import pallas as pl
from jax.experimental.pallas import tpu as pltpu
from jax.experimental.pallas import tpu_sc as plsc

N_CAT = 26
N_NUM = 13
K_PAD = 32
N_TOK = 1 + N_CAT + N_NUM
G = 8
L = 16


def _sc_info():
    try:
        info = plsc.get_sparse_core_info()
        return info.num_cores, info.num_subcores
    except Exception:
        return 2, 16


@functools.partial(jax.jit, static_argnums=(6, 7, 8))
def _tokenize(cat_t, num_t, emb_flat, w_flat, b_flat, cls_flat, B, D, V):
    NC, NS = _sc_info()
    NW = NC * NS
    assert B % (NW * G) == 0
    b_per_w = B // NW
    n_groups = b_per_w // G
    KD16 = D // L

    mesh = plsc.VectorSubcoreMesh(core_axis_name="c", subcore_axis_name="s")

    @functools.partial(
        pl.kernel,
        out_type=jax.ShapeDtypeStruct((B * N_TOK, D), jnp.float32),
        mesh=mesh,
        scratch_types=[
            pltpu.VMEM((N_CAT, b_per_w), jnp.int32),
            pltpu.VMEM((N_NUM, b_per_w), jnp.float32),
            pltpu.VMEM((b_per_w * K_PAD,), jnp.int32),
            pltpu.VMEM((N_NUM * D,), jnp.float32),
            pltpu.VMEM((N_NUM * D,), jnp.float32),
            pltpu.VMEM((D,), jnp.float32),
            pltpu.VMEM((G * N_TOK, D), jnp.float32),
            pltpu.VMEM((G * N_TOK, D), jnp.float32),
            pltpu.SemaphoreType.DMA,
            pltpu.SemaphoreType.DMA,
            pltpu.SemaphoreType.DMA,
        ],
        compiler_params=pltpu.CompilerParams(
            use_tc_tiling_on_sc=False, needs_layout_passes=False),
    )
    def k(cat_hbm, num_hbm, emb_hbm, w_hbm, bias_hbm, cls_hbm, out_hbm,
          cat_v, num_v, idx_v, w_v, bias_v, cls_v, buf0, buf1,
          sem_g, sem_w0, sem_w1):
        c = lax.axis_index("c")
        s = lax.axis_index("s")
        wid = s * NC + c
        base = wid * b_per_w

        pltpu.sync_copy(cat_hbm.at[:, pl.ds(base, b_per_w)], cat_v)
        pltpu.sync_copy(num_hbm.at[:, pl.ds(base, b_per_w)], num_v)
        pltpu.sync_copy(w_hbm, w_v)
        pltpu.sync_copy(bias_hbm, bias_v)
        pltpu.sync_copy(cls_hbm, cls_v)

        lanes = lax.iota(jnp.int32, L)
        for j in range(N_CAT):
            for g in range(b_per_w // L):
                vals = cat_v[j, pl.ds(g * L, L)] + jnp.int32(j * V)
                pos = (g * L + lanes) * K_PAD + j
                plsc.store_scatter(idx_v, [pos], vals)

        bufs = (buf0, buf1)
        wsems = (sem_w0, sem_w1)

        def group(g, carry):
            row0 = g * G
            slot = lax.rem(g, 2)

            def do_slot(buf, sem_w, sl):
                @pl.when(g >= 2)
                def _():
                    pltpu.make_async_copy(
                        buf,
                        out_hbm.at[pl.ds((base + (g - 2) * G) * N_TOK,
                                         G * N_TOK)],
                        sem_w,
                    ).wait()

                copies = []
                for r in range(G):
                    lr = row0 + r
                    copies.append(pltpu.async_copy(
                        emb_hbm.at[idx_v.at[pl.ds(lr * K_PAD, N_CAT)]],
                        buf.at[pl.ds(r * N_TOK + 1, N_CAT)],
                        sem_g,
                    ))
                for r in range(G):
                    lr = row0 + r
                    o = r * N_TOK
                    for kk in range(KD16):
                        buf[o, pl.ds(kk * L, L)] = cls_v[pl.ds(kk * L, L)]
                    for j in range(N_NUM):
                        nb = plsc.load_gather(
                            num_v,
                            [jnp.full((L,), j, jnp.int32),
                             jnp.full((L,), lr, jnp.int32)])
                        for kk in range(KD16):
                            wv = w_v[pl.ds((j * D) + kk * L, L)]
                            bv = bias_v[pl.ds((j * D) + kk * L, L)]
                            buf[o + 1 + N_CAT + j, pl.ds(kk * L, L)] = (
                                nb * wv + bv)
                for cp in copies:
                    cp.wait()
                pltpu.async_copy(
                    buf,
                    out_hbm.at[pl.ds((base + row0) * N_TOK, G * N_TOK)],
                    sem_w,
                )

            @pl.when(slot == 0)
            def _():
                do_slot(buf0, sem_w0, 0)

            @pl.when(slot == 1)
            def _():
                do_slot(buf1, sem_w1, 1)

            return carry

        lax.fori_loop(0, n_groups, group, 0)

        pltpu.make_async_copy(
            buf0,
            out_hbm.at[pl.ds((base + (n_groups - 2) * G) * N_TOK, G * N_TOK)],
            sem_w0,
        ).wait()
        pltpu.make_async_copy(
            buf1,
            out_hbm.at[pl.ds((base + (n_groups - 1) * G) * N_TOK, G * N_TOK)],
            sem_w1,
        ).wait()

    return k(cat_t, num_t, emb_flat, w_flat, b_flat, cls_flat)


def kernel(cat, num, emb_cat, w_num, b_num, cls):
    B = cat.shape[0]
    n_cat, vocab, D = emb_cat.shape
    out = _tokenize(
        cat.T,
        num.T,
        emb_cat.reshape(n_cat * vocab, D),
        w_num.reshape(-1),
        b_num.reshape(-1),
        cls.reshape(-1),
        B, D, vocab,
    )
    return out.reshape(B, N_TOK, D)


# speedup vs baseline: 2.8153x; 2.7928x over previous
import functools
import jax
import jax.numpy as jnp
from jax import lax
from jax.experimental import pallas as pl
from jax.experimental.pallas import tpu as pltpu
from jax.experimental.pallas import tpu_sc as plsc

N_CAT = 26
N_NUM = 13
N_TOK = 1 + N_CAT + N_NUM
L = 16
CHV = 4096
B_SHIFT = 17


def _sc_info():
    try:
        info = plsc.get_sparse_core_info()
        return info.num_cores, info.num_subcores
    except Exception:
        return 2, 16


@functools.partial(jax.jit, static_argnums=(7, 8, 9))
def _tokenize(cat_t, num_t, emb_t, tail_t, w_flat, b_flat, cls_flat, B, D, V):
    NC, NS = _sc_info()
    NW = NC * NS
    assert NW >= N_CAT + 1
    VF = V // 128 * 128
    TW = V - VF
    n_full = VF // CHV
    v_rem = VF - n_full * CHV
    chunks = [(kk * CHV, CHV) for kk in range(n_full)]
    if v_rem:
        chunks.append((n_full * CHV, v_rem))
    n_bkt = len(chunks) + (1 if TW else 0)
    bq_cap = (B + n_bkt * L + L - 1) // L * L
    n_groups = B // L
    DT = D // 8

    mesh = plsc.VectorSubcoreMesh(core_axis_name="c", subcore_axis_name="s")

    @functools.partial(
        pl.kernel,
        out_type=jax.ShapeDtypeStruct((N_TOK, D, B), jnp.float32),
        mesh=mesh,
        scratch_types=[
            pltpu.VMEM((B,), jnp.int32),
            pltpu.VMEM((bq_cap,), jnp.int32),
            pltpu.VMEM((8, CHV), jnp.float32),
            pltpu.VMEM((8, B + 128), jnp.float32),
            pltpu.VMEM((max(D * TW, 1),), jnp.float32),
            pltpu.VMEM((B,), jnp.float32),
            pltpu.VMEM((N_NUM * 64,), jnp.float32),
            pltpu.VMEM((N_NUM * 64,), jnp.float32),
            pltpu.VMEM((64,), jnp.float32),
        ],
        compiler_params=pltpu.CompilerParams(
            use_tc_tiling_on_sc=True, needs_layout_passes=False),
    )
    def k(cat_hbm, num_hbm, emb_hbm, tail_hbm, w_hbm, bias_hbm, cls_hbm,
          out_hbm, qv, bq, cbuf, res, tailv, nrow, wv, bv, clsv):
        c = lax.axis_index("c")
        s_ax = lax.axis_index("s")
        wid = s_ax * NC + c
        lanes = lax.iota(jnp.int32, L)

        @pl.when(wid < N_CAT)
        def column_worker():
            i = wid
            pltpu.sync_copy(cat_hbm.at[i], qv)
            if TW:
                pltpu.sync_copy(tail_hbm.at[pl.ds(i * D * TW, D * TW)], tailv)

            boffs = []
            pos = jnp.int32(0)
            for kk in range(n_bkt):
                boffs.append(pos)
                if TW and kk == n_bkt - 1:
                    lo, hi = VF, V
                else:
                    lo = chunks[kk][0]
                    hi = lo + chunks[kk][1]

                def scan(g, p, lo=lo, hi=hi):
                    v = qv[pl.ds(g * L, L)]
                    m = (v >= lo) & (v < hi)
                    inc = jnp.where(m, jnp.int32(1), jnp.int32(0))
                    cum = plsc.cumsum(inc)
                    packed = v | ((g * L + lanes) << B_SHIFT)
                    plsc.store_scatter(bq, [p + cum - 1], packed, mask=m)
                    return p + jnp.sum(inc)

                pos = lax.fori_loop(0, n_groups, scan, pos)
                npad = (-pos) & (L - 1)
                mpad = lanes < npad
                dummy = jnp.int32(lo) | ((B + lanes) << B_SHIFT)
                plsc.store_scatter(bq, [pos + lanes], dummy, mask=mpad)
                pos = pos + npad
            boffs.append(pos)

            def dt_body(dt, carry):
                for kk, (vlo, vlen) in enumerate(chunks):
                    pltpu.sync_copy(
                        emb_hbm.at[i, pl.ds(dt * 8, 8), pl.ds(vlo, vlen)],
                        cbuf.at[:, pl.ds(0, vlen)],
                    )

                    def extract(t, cc, vlo=vlo):
                        p = bq[pl.ds(t * L, L)]
                        v = p & ((1 << B_SHIFT) - 1)
                        b = lax.shift_right_logical(p, B_SHIFT)
                        vin = v - vlo
                        for ss in range(8):
                            vals = plsc.load_gather(
                                cbuf, [jnp.full((L,), ss, jnp.int32), vin])
                            plsc.store_scatter(
                                res, [jnp.full((L,), ss, jnp.int32), b], vals)
                        return cc

                    lax.fori_loop(boffs[kk] // L, boffs[kk + 1] // L,
                                  extract, 0)
                if TW:
                    kk = n_bkt - 1

                    def extract_tail(t, cc):
                        p = bq[pl.ds(t * L, L)]
                        v = p & ((1 << B_SHIFT) - 1)
                        b = lax.shift_right_logical(p, B_SHIFT)
                        vin = v - VF
                        for ss in range(8):
                            gidx = (dt * 8 + ss) * TW + vin
                            vals = plsc.load_gather(tailv, [gidx])
                            plsc.store_scatter(
                                res, [jnp.full((L,), ss, jnp.int32), b], vals)
                        return cc

                    lax.fori_loop(boffs[kk] // L, boffs[kk + 1] // L,
                                  extract_tail, 0)
                pltpu.sync_copy(
                    res.at[:, pl.ds(0, B)],
                    out_hbm.at[1 + i, pl.ds(dt * 8, 8), :])
                return carry

            lax.fori_loop(0, DT, dt_body, 0)

        @pl.when(wid == N_CAT)
        def cls_worker():
            pltpu.sync_copy(cls_hbm, clsv)

            def dt_body(dt, carry):
                for ss in range(8):
                    cvec = jnp.full((L,), ss, jnp.int32) + dt * 8
                    cval = plsc.load_gather(clsv, [cvec])

                    def fill(g, cc, ss=ss, cval=cval):
                        res[ss, pl.ds(g * L, L)] = cval
                        return cc

                    lax.fori_loop(0, n_groups, fill, 0)
                pltpu.sync_copy(
                    res.at[:, pl.ds(0, B)],
                    out_hbm.at[0, pl.ds(dt * 8, 8), :])
                return carry

            lax.fori_loop(0, DT, dt_body, 0)

        @pl.when(wid > N_CAT)
        def num_worker():
            r = wid - (N_CAT + 1)
            n_aux = NW - (N_CAT + 1)
            pltpu.sync_copy(w_hbm, wv)
            pltpu.sync_copy(bias_hbm, bv)
            n_units = N_NUM * DT

            def unit_body(u, carry):
                unit = r + u * n_aux

                @pl.when(unit < n_units)
                def _():
                    j = unit // DT
                    dt = lax.rem(unit, DT)
                    pltpu.sync_copy(num_hbm.at[j], nrow)
                    for ss in range(8):
                        widx = jnp.full((L,), ss, jnp.int32) + (j * 64 + dt * 8)
                        wval = plsc.load_gather(wv, [widx])
                        bval = plsc.load_gather(bv, [widx])

                        def fill(g, cc, ss=ss, wval=wval, bval=bval):
                            x = nrow[pl.ds(g * L, L)]
                            res[ss, pl.ds(g * L, L)] = x * wval + bval
                            return cc

                        lax.fori_loop(0, n_groups, fill, 0)
                    pltpu.sync_copy(
                        res.at[:, pl.ds(0, B)],
                        out_hbm.at[1 + N_CAT + j, pl.ds(dt * 8, 8), :])

                return carry

            lax.fori_loop(0, (n_units + n_aux - 1) // n_aux, unit_body, 0)

    return k(cat_t, num_t, emb_t, tail_t, w_flat, b_flat, cls_flat)


def kernel(cat, num, emb_cat, w_num, b_num, cls):
    B = cat.shape[0]
    n_cat, vocab, D = emb_cat.shape
    VF = vocab // 128 * 128
    tail_t = jnp.transpose(emb_cat[:, VF:, :], (0, 2, 1)).reshape(-1)
    out_k = _tokenize(
        cat.T,
        num.T,
        jnp.transpose(emb_cat, (0, 2, 1)),
        tail_t,
        w_num.reshape(-1),
        b_num.reshape(-1),
        cls.reshape(-1),
        B, D, vocab,
    )
    return jnp.transpose(out_k, (2, 0, 1))


# speedup vs baseline: 2.8922x; 1.0273x over previous
import functools
import jax
import jax.numpy as jnp
from jax import lax
from jax.experimental import pallas as pl
from jax.experimental.pallas import tpu as pltpu
from jax.experimental.pallas import tpu_sc as plsc

N_CAT = 26
N_NUM = 13
N_TOK = 1 + N_CAT + N_NUM
L = 16
CHV = 4096
B_SHIFT = 17


def _sc_info():
    try:
        info = plsc.get_sparse_core_info()
        return info.num_cores, info.num_subcores
    except Exception:
        return 2, 16


@functools.partial(jax.jit, static_argnums=(7, 8, 9))
def _tokenize(cat_t, num_t, emb_t, tail_t, w_flat, b_flat, cls_flat, B, D, V):
    NC, NS = _sc_info()
    NW = NC * NS
    VF = V // 128 * 128
    TW = V - VF
    n_full = VF // CHV
    v_rem = VF - n_full * CHV
    chunks = [(kk * CHV, CHV) for kk in range(n_full)]
    if v_rem:
        chunks.append((n_full * CHV, v_rem))
    n_chunks = len(chunks)
    n_bkt = n_chunks + (1 if TW else 0)
    bq_cap = (B + n_bkt * L + L - 1) // L * L
    n_groups = B // L
    DT = D // 8
    n_units = N_CAT * DT
    n_aux = DT + N_NUM * DT

    mesh = plsc.VectorSubcoreMesh(core_axis_name="c", subcore_axis_name="s")

    @functools.partial(
        pl.kernel,
        out_type=jax.ShapeDtypeStruct((N_TOK, D, B), jnp.float32),
        mesh=mesh,
        scratch_types=[
            pltpu.VMEM((B,), jnp.int32),
            pltpu.VMEM((bq_cap,), jnp.int32),
            pltpu.VMEM((8, CHV), jnp.float32),
            pltpu.VMEM((8, CHV), jnp.float32),
            pltpu.VMEM((8, B + 128), jnp.float32),
            pltpu.VMEM((max(D * TW, 1),), jnp.float32),
            pltpu.VMEM((B,), jnp.float32),
            pltpu.VMEM((N_NUM * 64,), jnp.float32),
            pltpu.VMEM((N_NUM * 64,), jnp.float32),
            pltpu.VMEM((64,), jnp.float32),
            pltpu.SemaphoreType.DMA,
            pltpu.SemaphoreType.DMA,
        ],
        compiler_params=pltpu.CompilerParams(
            use_tc_tiling_on_sc=True, needs_layout_passes=False),
    )
    def k(cat_hbm, num_hbm, emb_hbm, tail_hbm, w_hbm, bias_hbm, cls_hbm,
          out_hbm, qv, bq, cbuf0, cbuf1, res, tailv, nrow, wv, bv, clsv,
          sem0, sem1):
        c = lax.axis_index("c")
        s_ax = lax.axis_index("s")
        wid = s_ax * NC + c
        lanes = lax.iota(jnp.int32, L)

        def stream_phase(i, dlo, dhi):
            pltpu.sync_copy(cat_hbm.at[i], qv)
            if TW:
                pltpu.sync_copy(tail_hbm.at[pl.ds(i * D * TW, D * TW)], tailv)

            boffs = []
            pos = jnp.int32(0)
            for kk in range(n_bkt):
                boffs.append(pos)
                if TW and kk == n_bkt - 1:
                    lo, hi = VF, V
                else:
                    lo = chunks[kk][0]
                    hi = lo + chunks[kk][1]

                def scan(g, p, lo=lo, hi=hi):
                    v = qv[pl.ds(g * L, L)]
                    m = (v >= lo) & (v < hi)
                    inc = jnp.where(m, jnp.int32(1), jnp.int32(0))
                    cum = plsc.cumsum(inc)
                    packed = v | ((g * L + lanes) << B_SHIFT)
                    plsc.store_scatter(bq, [p + cum - 1], packed, mask=m)
                    return p + jnp.sum(inc)

                pos = lax.fori_loop(0, n_groups, scan, pos)
                npad = (-pos) & (L - 1)
                mpad = lanes < npad
                dummy = jnp.int32(lo) | ((B + lanes) << B_SHIFT)
                plsc.store_scatter(bq, [pos + lanes], dummy, mask=mpad)
                pos = pos + npad
            boffs.append(pos)

            def dt_body(dt, carry):
                bufs = (cbuf0, cbuf1)
                sems = (sem0, sem1)

                def start(kk):
                    vlo, vlen = chunks[kk]
                    return pltpu.async_copy(
                        emb_hbm.at[i, pl.ds(dt * 8, 8), pl.ds(vlo, vlen)],
                        bufs[kk % 2].at[:, pl.ds(0, vlen)],
                        sems[kk % 2],
                    )

                pending = start(0)
                for kk, (vlo, vlen) in enumerate(chunks):
                    nxt = start(kk + 1) if kk + 1 < n_chunks else None
                    pending.wait()
                    cur = bufs[kk % 2]

                    def extract(t, cc, vlo=vlo, cur=cur):
                        p = bq[pl.ds(t * L, L)]
                        v = p & ((1 << B_SHIFT) - 1)
                        b = lax.shift_right_logical(p, B_SHIFT)
                        vin = v - vlo
                        for ss in range(8):
                            vals = plsc.load_gather(
                                cur, [jnp.full((L,), ss, jnp.int32), vin])
                            plsc.store_scatter(
                                res, [jnp.full((L,), ss, jnp.int32), b], vals)
                        return cc

                    lax.fori_loop(boffs[kk] // L, boffs[kk + 1] // L,
                                  extract, 0)
                    pending = nxt
                if TW:
                    kk = n_bkt - 1

                    def extract_tail(t, cc):
                        p = bq[pl.ds(t * L, L)]
                        v = p & ((1 << B_SHIFT) - 1)
                        b = lax.shift_right_logical(p, B_SHIFT)
                        vin = v - VF
                        for ss in range(8):
                            gidx = (dt * 8 + ss) * TW + vin
                            vals = plsc.load_gather(tailv, [gidx])
                            plsc.store_scatter(
                                res, [jnp.full((L,), ss, jnp.int32), b], vals)
                        return cc

                    lax.fori_loop(boffs[kk] // L, boffs[kk + 1] // L,
                                  extract_tail, 0)
                pltpu.sync_copy(
                    res.at[:, pl.ds(0, B)],
                    out_hbm.at[1 + i, pl.ds(dt * 8, 8), :])
                return carry

            lax.fori_loop(dlo, dhi, dt_body, 0)

        base = (wid * n_units) // NW
        cnt = ((wid + 1) * n_units) // NW - base
        iA = base // DT
        dloA = lax.rem(base, DT)
        dhiA = jnp.minimum(jnp.int32(DT), dloA + cnt)

        @pl.when(dhiA > dloA)
        def phase_a():
            stream_phase(iA, dloA, dhiA)

        iB = iA + 1
        dhiB = jnp.minimum((base + cnt) - iB * DT, jnp.int32(DT))

        @pl.when((dhiB > 0) & (iB < N_CAT))
        def phase_b():
            stream_phase(iB, jnp.int32(0), dhiB)

        pltpu.sync_copy(w_hbm, wv)
        pltpu.sync_copy(bias_hbm, bv)
        pltpu.sync_copy(cls_hbm, clsv)

        def aux_body(u, carry):
            aux = wid + u * NW

            @pl.when(aux < DT)
            def cls_block():
                dt = aux
                for ss in range(8):
                    cvec = jnp.full((L,), ss, jnp.int32) + dt * 8
                    cval = plsc.load_gather(clsv, [cvec])

                    def fill(g, cc, ss=ss, cval=cval):
                        res[ss, pl.ds(g * L, L)] = cval
                        return cc

                    lax.fori_loop(0, n_groups, fill, 0)
                pltpu.sync_copy(
                    res.at[:, pl.ds(0, B)],
                    out_hbm.at[0, pl.ds(dt * 8, 8), :])

            @pl.when((aux >= DT) & (aux < n_aux))
            def num_block():
                j = (aux - DT) // DT
                dt = lax.rem(aux - DT, DT)
                pltpu.sync_copy(num_hbm.at[j], nrow)
                for ss in range(8):
                    widx = jnp.full((L,), ss, jnp.int32) + (j * 64 + dt * 8)
                    wval = plsc.load_gather(wv, [widx])
                    bval = plsc.load_gather(bv, [widx])

                    def fill(g, cc, ss=ss, wval=wval, bval=bval):
                        x = nrow[pl.ds(g * L, L)]
                        res[ss, pl.ds(g * L, L)] = x * wval + bval
                        return cc

                    lax.fori_loop(0, n_groups, fill, 0)
                pltpu.sync_copy(
                    res.at[:, pl.ds(0, B)],
                    out_hbm.at[1 + N_CAT + j, pl.ds(dt * 8, 8), :])

            return carry

        lax.fori_loop(0, (n_aux + NW - 1) // NW, aux_body, 0)

    return k(cat_t, num_t, emb_t, tail_t, w_flat, b_flat, cls_flat)


def kernel(cat, num, emb_cat, w_num, b_num, cls):
    B = cat.shape[0]
    n_cat, vocab, D = emb_cat.shape
    VF = vocab // 128 * 128
    tail_t = jnp.transpose(emb_cat[:, VF:, :], (0, 2, 1)).reshape(-1)
    out_k = _tokenize(
        cat.T,
        num.T,
        jnp.transpose(emb_cat, (0, 2, 1)),
        tail_t,
        w_num.reshape(-1),
        b_num.reshape(-1),
        cls.reshape(-1),
        B, D, vocab,
    )
    return jnp.transpose(out_k, (2, 0, 1))


# speedup vs baseline: 3.1363x; 1.0844x over previous
import functools
import jax
import jax.numpy as jnp
from jax import lax
from jax.experimental import pallas as pl
from jax.experimental.pallas import tpu as pltpu
from jax.experimental.pallas import tpu_sc as plsc

N_CAT = 26
N_NUM = 13
N_TOK = 1 + N_CAT + N_NUM
L = 16
CHV = 8192
CH_SHIFT = 13
B_SHIFT = 17


def _sc_info():
    try:
        info = plsc.get_sparse_core_info()
        return info.num_cores, info.num_subcores
    except Exception:
        return 2, 16


@functools.partial(jax.jit, static_argnums=(7, 8, 9))
def _tokenize(cat_t, num_t, emb_t, tail_t, w_flat, b_flat, cls_flat, B, D, V):
    NC, NS = _sc_info()
    NW = NC * NS
    VF = V // 128 * 128
    TW = V - VF
    n_full = VF // CHV
    v_rem = VF - n_full * CHV
    chunks = [(kk * CHV, CHV) for kk in range(n_full)]
    if v_rem:
        chunks.append((n_full * CHV, v_rem))
    n_chunks = len(chunks)
    n_bkt = n_chunks + (1 if TW else 0)
    bq_cap = (B + n_bkt * L + L - 1) // L * L
    n_groups = B // L
    DT = D // 8
    n_units = N_CAT * DT
    n_aux = DT + N_NUM * DT

    mesh = plsc.VectorSubcoreMesh(core_axis_name="c", subcore_axis_name="s")

    @functools.partial(
        pl.kernel,
        out_type=jax.ShapeDtypeStruct((N_TOK, D, B), jnp.float32),
        mesh=mesh,
        scratch_types=[
            pltpu.VMEM((B,), jnp.int32),
            pltpu.VMEM((bq_cap,), jnp.int32),
            pltpu.VMEM((8, CHV), jnp.float32),
            pltpu.VMEM((8, B + 128), jnp.float32),
            pltpu.VMEM((max(D * TW, 1),), jnp.float32),
            pltpu.VMEM((B,), jnp.float32),
            pltpu.VMEM((N_NUM * 64,), jnp.float32),
            pltpu.VMEM((N_NUM * 64,), jnp.float32),
            pltpu.VMEM((64,), jnp.float32),
        ],
        compiler_params=pltpu.CompilerParams(
            use_tc_tiling_on_sc=True, needs_layout_passes=False),
    )
    def k(cat_hbm, num_hbm, emb_hbm, tail_hbm, w_hbm, bias_hbm, cls_hbm,
          out_hbm, qv, bq, cbuf, res, tailv, nrow, wv, bv, clsv):
        c = lax.axis_index("c")
        s_ax = lax.axis_index("s")
        wid = s_ax * NC + c
        lanes = lax.iota(jnp.int32, L)

        def stream_phase(i, dlo, dhi):
            pltpu.sync_copy(cat_hbm.at[i], qv)
            if TW:
                pltpu.sync_copy(tail_hbm.at[pl.ds(i * D * TW, D * TW)], tailv)

            boffs = []
            pos = jnp.int32(0)
            for kk in range(n_bkt):
                boffs.append(pos)
                if TW and kk == n_bkt - 1:
                    lo, hi = VF, V
                else:
                    lo = chunks[kk][0]
                    hi = lo + chunks[kk][1]

                def mask_of(v, lo=lo, hi=hi, kk=kk):
                    if TW and kk == n_bkt - 1:
                        return v >= lo
                    m = lax.shift_right_logical(v, CH_SHIFT) == kk
                    if hi == VF and TW:
                        m = m & (v < VF)
                    return m

                def scan(g, p):
                    g2 = g * 2
                    v0 = qv[pl.ds(g2 * L, L)]
                    v1 = qv[pl.ds((g2 + 1) * L, L)]
                    m0 = mask_of(v0)
                    m1 = mask_of(v1)
                    inc0 = jnp.where(m0, jnp.int32(1), jnp.int32(0))
                    inc1 = jnp.where(m1, jnp.int32(1), jnp.int32(0))
                    cum0 = plsc.cumsum(inc0)
                    cum1 = plsc.cumsum(inc1)
                    t0 = jnp.sum(inc0)
                    t1 = jnp.sum(inc1)
                    p0 = v0 | ((g2 * L + lanes) << B_SHIFT)
                    p1 = v1 | (((g2 + 1) * L + lanes) << B_SHIFT)
                    plsc.store_scatter(bq, [p + cum0 - 1], p0, mask=m0)
                    plsc.store_scatter(bq, [p + t0 + cum1 - 1], p1, mask=m1)
                    return p + t0 + t1

                pos = lax.fori_loop(0, n_groups // 2, scan, pos)
                npad = (-pos) & (L - 1)
                mpad = lanes < npad
                dummy = jnp.int32(lo) | ((B + lanes) << B_SHIFT)
                plsc.store_scatter(bq, [pos + lanes], dummy, mask=mpad)
                pos = pos + npad
            boffs.append(pos)

            def dt_body(dt, carry):
                for kk, (vlo, vlen) in enumerate(chunks):
                    pltpu.sync_copy(
                        emb_hbm.at[i, pl.ds(dt * 8, 8), pl.ds(vlo, vlen)],
                        cbuf.at[:, pl.ds(0, vlen)],
                    )

                    def extract(t, cc, vlo=vlo, cur=cbuf):
                        p = bq[pl.ds(t * L, L)]
                        v = p & ((1 << B_SHIFT) - 1)
                        b = lax.shift_right_logical(p, B_SHIFT)
                        vin = v - vlo
                        for ss in range(8):
                            vals = plsc.load_gather(
                                cur, [jnp.full((L,), ss, jnp.int32), vin])
                            plsc.store_scatter(
                                res, [jnp.full((L,), ss, jnp.int32), b], vals)
                        return cc

                    lax.fori_loop(boffs[kk] // L, boffs[kk + 1] // L,
                                  extract, 0)
                if TW:
                    kk = n_bkt - 1

                    def extract_tail(t, cc):
                        p = bq[pl.ds(t * L, L)]
                        v = p & ((1 << B_SHIFT) - 1)
                        b = lax.shift_right_logical(p, B_SHIFT)
                        vin = v - VF
                        for ss in range(8):
                            gidx = (dt * 8 + ss) * TW + vin
                            vals = plsc.load_gather(tailv, [gidx])
                            plsc.store_scatter(
                                res, [jnp.full((L,), ss, jnp.int32), b], vals)
                        return cc

                    lax.fori_loop(boffs[kk] // L, boffs[kk + 1] // L,
                                  extract_tail, 0)
                pltpu.sync_copy(
                    res.at[:, pl.ds(0, B)],
                    out_hbm.at[1 + i, pl.ds(dt * 8, 8), :])
                return carry

            lax.fori_loop(dlo, dhi, dt_body, 0)

        base = (wid * n_units) // NW
        cnt = ((wid + 1) * n_units) // NW - base
        iA = base // DT
        dloA = lax.rem(base, DT)
        dhiA = jnp.minimum(jnp.int32(DT), dloA + cnt)

        @pl.when(dhiA > dloA)
        def phase_a():
            stream_phase(iA, dloA, dhiA)

        iB = iA + 1
        dhiB = jnp.minimum((base + cnt) - iB * DT, jnp.int32(DT))

        @pl.when((dhiB > 0) & (iB < N_CAT))
        def phase_b():
            stream_phase(iB, jnp.int32(0), dhiB)

        pltpu.sync_copy(w_hbm, wv)
        pltpu.sync_copy(bias_hbm, bv)
        pltpu.sync_copy(cls_hbm, clsv)

        def aux_body(u, carry):
            aux = wid + u * NW

            @pl.when(aux < DT)
            def cls_block():
                dt = aux
                for ss in range(8):
                    cvec = jnp.full((L,), ss, jnp.int32) + dt * 8
                    cval = plsc.load_gather(clsv, [cvec])

                    def fill(g, cc, ss=ss, cval=cval):
                        res[ss, pl.ds(g * L, L)] = cval
                        return cc

                    lax.fori_loop(0, n_groups, fill, 0)
                pltpu.sync_copy(
                    res.at[:, pl.ds(0, B)],
                    out_hbm.at[0, pl.ds(dt * 8, 8), :])

            @pl.when((aux >= DT) & (aux < n_aux))
            def num_block():
                j = (aux - DT) // DT
                dt = lax.rem(aux - DT, DT)
                pltpu.sync_copy(num_hbm.at[j], nrow)
                for ss in range(8):
                    widx = jnp.full((L,), ss, jnp.int32) + (j * 64 + dt * 8)
                    wval = plsc.load_gather(wv, [widx])
                    bval = plsc.load_gather(bv, [widx])

                    def fill(g, cc, ss=ss, wval=wval, bval=bval):
                        x = nrow[pl.ds(g * L, L)]
                        res[ss, pl.ds(g * L, L)] = x * wval + bval
                        return cc

                    lax.fori_loop(0, n_groups, fill, 0)
                pltpu.sync_copy(
                    res.at[:, pl.ds(0, B)],
                    out_hbm.at[1 + N_CAT + j, pl.ds(dt * 8, 8), :])

            return carry

        lax.fori_loop(0, (n_aux + NW - 1) // NW, aux_body, 0)

    return k(cat_t, num_t, emb_t, tail_t, w_flat, b_flat, cls_flat)


def kernel(cat, num, emb_cat, w_num, b_num, cls):
    B = cat.shape[0]
    n_cat, vocab, D = emb_cat.shape
    VF = vocab // 128 * 128
    tail_t = jnp.transpose(emb_cat[:, VF:, :], (0, 2, 1)).reshape(-1)
    out_k = _tokenize(
        cat.T,
        num.T,
        jnp.transpose(emb_cat, (0, 2, 1)),
        tail_t,
        w_num.reshape(-1),
        b_num.reshape(-1),
        cls.reshape(-1),
        B, D, vocab,
    )
    return jnp.transpose(out_k, (2, 0, 1))


# speedup vs baseline: 3.3745x; 1.0759x over previous
import functools
import jax
import jax.numpy as jnp
from jax import lax
from jax.experimental import pallas as pl
from jax.experimental.pallas import tpu as pltpu
from jax.experimental.pallas import tpu_sc as plsc

N_CAT = 26
N_NUM = 13
N_TOK = 1 + N_CAT + N_NUM
L = 16
CHV = 8192
CH_SHIFT = 13
B_SHIFT = 17


def _sc_info():
    try:
        info = plsc.get_sparse_core_info()
        return info.num_cores, info.num_subcores
    except Exception:
        return 2, 16


@functools.partial(jax.jit, static_argnums=(7, 8, 9))
def _tokenize(cat_t, num_t, emb_t, tail_t, w_flat, b_flat, cls_flat, B, D, V):
    NC, NS = _sc_info()
    NW = NC * NS
    VF = V // 128 * 128
    TW = V - VF
    n_full = VF // CHV
    v_rem = VF - n_full * CHV
    chunks = [(kk * CHV, CHV) for kk in range(n_full)]
    if v_rem:
        chunks.append((n_full * CHV, v_rem))
    n_chunks = len(chunks)
    n_bkt = n_chunks + (1 if TW else 0)
    bq_cap = (B + n_bkt * L + L - 1) // L * L
    n_groups = B // L
    DT = D // 8
    n_units = N_CAT * DT
    n_aux = DT + N_NUM * DT

    mesh = plsc.VectorSubcoreMesh(core_axis_name="c", subcore_axis_name="s")

    @functools.partial(
        pl.kernel,
        out_type=jax.ShapeDtypeStruct((N_TOK, D, B), jnp.float32),
        mesh=mesh,
        scratch_types=[
            pltpu.VMEM((B,), jnp.int32),
            pltpu.VMEM((bq_cap,), jnp.int32),
            pltpu.VMEM((8, CHV), jnp.float32),
            pltpu.VMEM((8, B + 128), jnp.float32),
            pltpu.VMEM((max(D * TW, 1),), jnp.float32),
            pltpu.VMEM((B,), jnp.float32),
            pltpu.VMEM((N_NUM * 64,), jnp.float32),
            pltpu.VMEM((N_NUM * 64,), jnp.float32),
            pltpu.VMEM((64,), jnp.float32),
        ],
        compiler_params=pltpu.CompilerParams(
            use_tc_tiling_on_sc=True, needs_layout_passes=False),
    )
    def k(cat_hbm, num_hbm, emb_hbm, tail_hbm, w_hbm, bias_hbm, cls_hbm,
          out_hbm, qv, bq, cbuf, res, tailv, nrow, wv, bv, clsv):
        c = lax.axis_index("c")
        s_ax = lax.axis_index("s")
        wid = s_ax * NC + c
        lanes = lax.iota(jnp.int32, L)

        def stream_phase(i, dlo, dhi):
            pltpu.sync_copy(cat_hbm.at[i], qv)
            if TW:
                pltpu.sync_copy(tail_hbm.at[pl.ds(i * D * TW, D * TW)], tailv)

            boffs = []
            pos = jnp.int32(0)
            for kk in range(n_bkt):
                boffs.append(pos)
                if TW and kk == n_bkt - 1:
                    lo, hi = VF, V
                else:
                    lo = chunks[kk][0]
                    hi = lo + chunks[kk][1]

                def mask_of(v, lo=lo, hi=hi, kk=kk):
                    if TW and kk == n_bkt - 1:
                        return v >= lo
                    m = lax.shift_right_logical(v, CH_SHIFT) == kk
                    if hi == VF and TW:
                        m = m & (v < VF)
                    return m

                def scan(g, p):
                    g4 = g * 4
                    vs = [qv[pl.ds((g4 + r) * L, L)] for r in range(4)]
                    ms = [mask_of(v) for v in vs]
                    incs = [jnp.where(m, jnp.int32(1), jnp.int32(0))
                            for m in ms]
                    cums = [plsc.cumsum(inc) for inc in incs]
                    ts = [jnp.sum(inc) for inc in incs]
                    pk = [vs[r] | (((g4 + r) * L + lanes) << B_SHIFT)
                          for r in range(4)]
                    off = p
                    for r in range(4):
                        plsc.store_scatter(bq, [off + cums[r] - 1], pk[r],
                                           mask=ms[r])
                        off = off + ts[r]
                    return off

                pos = lax.fori_loop(0, n_groups // 4, scan, pos)
                npad = (-pos) & (L - 1)
                mpad = lanes < npad
                dummy = jnp.int32(lo) | ((B + lanes) << B_SHIFT)
                plsc.store_scatter(bq, [pos + lanes], dummy, mask=mpad)
                pos = pos + npad
            boffs.append(pos)

            def dt_body(dt, carry):
                for kk, (vlo, vlen) in enumerate(chunks):
                    pltpu.sync_copy(
                        emb_hbm.at[i, pl.ds(dt * 8, 8), pl.ds(vlo, vlen)],
                        cbuf.at[:, pl.ds(0, vlen)],
                    )

                    def extract(t, cc, vlo=vlo, cur=cbuf):
                        p = bq[pl.ds(t * L, L)]
                        v = p & ((1 << B_SHIFT) - 1)
                        b = lax.shift_right_logical(p, B_SHIFT)
                        vin = v - vlo
                        for ss in range(8):
                            vals = plsc.load_gather(
                                cur, [jnp.full((L,), ss, jnp.int32), vin])
                            plsc.store_scatter(
                                res, [jnp.full((L,), ss, jnp.int32), b], vals)
                        return cc

                    lax.fori_loop(boffs[kk] // L, boffs[kk + 1] // L,
                                  extract, 0)
                if TW:
                    kk = n_bkt - 1

                    def extract_tail(t, cc):
                        p = bq[pl.ds(t * L, L)]
                        v = p & ((1 << B_SHIFT) - 1)
                        b = lax.shift_right_logical(p, B_SHIFT)
                        vin = v - VF
                        for ss in range(8):
                            gidx = (dt * 8 + ss) * TW + vin
                            vals = plsc.load_gather(tailv, [gidx])
                            plsc.store_scatter(
                                res, [jnp.full((L,), ss, jnp.int32), b], vals)
                        return cc

                    lax.fori_loop(boffs[kk] // L, boffs[kk + 1] // L,
                                  extract_tail, 0)
                pltpu.sync_copy(
                    res.at[:, pl.ds(0, B)],
                    out_hbm.at[1 + i, pl.ds(dt * 8, 8), :])
                return carry

            lax.fori_loop(dlo, dhi, dt_body, 0)

        base = (wid * n_units) // NW
        cnt = ((wid + 1) * n_units) // NW - base
        iA = base // DT
        dloA = lax.rem(base, DT)
        dhiA = jnp.minimum(jnp.int32(DT), dloA + cnt)

        @pl.when(dhiA > dloA)
        def phase_a():
            stream_phase(iA, dloA, dhiA)

        iB = iA + 1
        dhiB = jnp.minimum((base + cnt) - iB * DT, jnp.int32(DT))

        @pl.when((dhiB > 0) & (iB < N_CAT))
        def phase_b():
            stream_phase(iB, jnp.int32(0), dhiB)

        pltpu.sync_copy(w_hbm, wv)
        pltpu.sync_copy(bias_hbm, bv)
        pltpu.sync_copy(cls_hbm, clsv)

        def aux_body(u, carry):
            aux = wid + u * NW

            @pl.when(aux < DT)
            def cls_block():
                dt = aux
                for ss in range(8):
                    cvec = jnp.full((L,), ss, jnp.int32) + dt * 8
                    cval = plsc.load_gather(clsv, [cvec])

                    def fill(g, cc, ss=ss, cval=cval):
                        res[ss, pl.ds(g * L, L)] = cval
                        return cc

                    lax.fori_loop(0, n_groups, fill, 0)
                pltpu.sync_copy(
                    res.at[:, pl.ds(0, B)],
                    out_hbm.at[0, pl.ds(dt * 8, 8), :])

            @pl.when((aux >= DT) & (aux < n_aux))
            def num_block():
                j = (aux - DT) // DT
                dt = lax.rem(aux - DT, DT)
                pltpu.sync_copy(num_hbm.at[j], nrow)
                for ss in range(8):
                    widx = jnp.full((L,), ss, jnp.int32) + (j * 64 + dt * 8)
                    wval = plsc.load_gather(wv, [widx])
                    bval = plsc.load_gather(bv, [widx])

                    def fill(g, cc, ss=ss, wval=wval, bval=bval):
                        x = nrow[pl.ds(g * L, L)]
                        res[ss, pl.ds(g * L, L)] = x * wval + bval
                        return cc

                    lax.fori_loop(0, n_groups, fill, 0)
                pltpu.sync_copy(
                    res.at[:, pl.ds(0, B)],
                    out_hbm.at[1 + N_CAT + j, pl.ds(dt * 8, 8), :])

            return carry

        lax.fori_loop(0, (n_aux + NW - 1) // NW, aux_body, 0)

    return k(cat_t, num_t, emb_t, tail_t, w_flat, b_flat, cls_flat)


def kernel(cat, num, emb_cat, w_num, b_num, cls):
    B = cat.shape[0]
    n_cat, vocab, D = emb_cat.shape
    VF = vocab // 128 * 128
    tail_t = jnp.transpose(emb_cat[:, VF:, :], (0, 2, 1)).reshape(-1)
    out_k = _tokenize(
        cat.T,
        num.T,
        jnp.transpose(emb_cat, (0, 2, 1)),
        tail_t,
        w_num.reshape(-1),
        b_num.reshape(-1),
        cls.reshape(-1),
        B, D, vocab,
    )
    return jnp.transpose(out_k, (2, 0, 1))


# speedup vs baseline: 3.5470x; 1.0511x over previous
import functools
import jax
import jax.numpy as jnp
from jax import lax
from jax.experimental import pallas as pl
from jax.experimental.pallas import tpu as pltpu
from jax.experimental.pallas import tpu_sc as plsc

N_CAT = 26
N_NUM = 13
N_TOK = 1 + N_CAT + N_NUM
L = 16
CHV = 8192
CH_SHIFT = 13
B_SHIFT = 17


def _sc_info():
    try:
        info = plsc.get_sparse_core_info()
        return info.num_cores, info.num_subcores
    except Exception:
        return 2, 16


@functools.partial(jax.jit, static_argnums=(7, 8, 9))
def _tokenize(cat_t, num_t, emb_t, tail_t, w_flat, b_flat, cls_flat, B, D, V):
    NC, NS = _sc_info()
    NW = NC * NS
    VF = V // 128 * 128
    TW = V - VF
    n_full = VF // CHV
    v_rem = VF - n_full * CHV
    chunks = [(kk * CHV, CHV) for kk in range(n_full)]
    if v_rem:
        chunks.append((n_full * CHV, v_rem))
    n_chunks = len(chunks)
    n_bkt = n_chunks + (1 if TW else 0)
    bq_cap = (B + n_bkt * L + L - 1) // L * L
    n_groups = B // L
    DT = D // 8
    n_units = N_CAT * DT
    n_aux = DT + N_NUM * DT

    mesh = plsc.VectorSubcoreMesh(core_axis_name="c", subcore_axis_name="s")

    @functools.partial(
        pl.kernel,
        out_type=jax.ShapeDtypeStruct((N_TOK, D, B), jnp.float32),
        mesh=mesh,
        scratch_types=[
            pltpu.VMEM((B,), jnp.int32),
            pltpu.VMEM((bq_cap,), jnp.int32),
            pltpu.VMEM((8, CHV), jnp.float32),
            pltpu.VMEM((8, B + 128), jnp.float32),
            pltpu.VMEM((max(D * TW, 1),), jnp.float32),
            pltpu.VMEM((B,), jnp.float32),
            pltpu.VMEM((N_NUM * 64,), jnp.float32),
            pltpu.VMEM((N_NUM * 64,), jnp.float32),
            pltpu.VMEM((64,), jnp.float32),
        ],
        compiler_params=pltpu.CompilerParams(
            use_tc_tiling_on_sc=True, needs_layout_passes=False),
    )
    def k(cat_hbm, num_hbm, emb_hbm, tail_hbm, w_hbm, bias_hbm, cls_hbm,
          out_hbm, qv, bq, cbuf, res, tailv, nrow, wv, bv, clsv):
        c = lax.axis_index("c")
        s_ax = lax.axis_index("s")
        wid = s_ax * NC + c
        lanes = lax.iota(jnp.int32, L)

        def stream_phase(i, dlo, dhi):
            pltpu.sync_copy(cat_hbm.at[i], qv)
            if TW:
                pltpu.sync_copy(tail_hbm.at[pl.ds(i * D * TW, D * TW)], tailv)

            boffs = []
            pos = jnp.int32(0)
            for kk in range(n_bkt):
                boffs.append(pos)
                if TW and kk == n_bkt - 1:
                    lo, hi = VF, V
                else:
                    lo = chunks[kk][0]
                    hi = lo + chunks[kk][1]

                def mask_of(v, lo=lo, hi=hi, kk=kk):
                    if TW and kk == n_bkt - 1:
                        return v >= lo
                    m = lax.shift_right_logical(v, CH_SHIFT) == kk
                    if hi == VF and TW:
                        m = m & (v < VF)
                    return m

                def scan(g, p):
                    g4 = g * 16
                    vs = [qv[pl.ds((g4 + r) * L, L)] for r in range(16)]
                    ms = [mask_of(v) for v in vs]
                    incs = [jnp.where(m, jnp.int32(1), jnp.int32(0))
                            for m in ms]
                    cums = [plsc.cumsum(inc) for inc in incs]
                    ts = [jnp.sum(inc) for inc in incs]
                    pk = [vs[r] | (((g4 + r) * L + lanes) << B_SHIFT)
                          for r in range(16)]
                    off = p
                    for r in range(16):
                        plsc.store_scatter(bq, [off + cums[r] - 1], pk[r],
                                           mask=ms[r])
                        off = off + ts[r]
                    return off

                pos = lax.fori_loop(0, n_groups // 16, scan, pos)
                npad = (-pos) & (L - 1)
                mpad = lanes < npad
                dummy = jnp.int32(lo) | ((B + lanes) << B_SHIFT)
                plsc.store_scatter(bq, [pos + lanes], dummy, mask=mpad)
                pos = pos + npad
            boffs.append(pos)

            def dt_body(dt, carry):
                for kk, (vlo, vlen) in enumerate(chunks):
                    pltpu.sync_copy(
                        emb_hbm.at[i, pl.ds(dt * 8, 8), pl.ds(vlo, vlen)],
                        cbuf.at[:, pl.ds(0, vlen)],
                    )

                    def extract(t, cc, vlo=vlo, cur=cbuf):
                        p = bq[pl.ds(t * L, L)]
                        v = p & ((1 << B_SHIFT) - 1)
                        b = lax.shift_right_logical(p, B_SHIFT)
                        vin = v - vlo
                        for ss in range(8):
                            vals = plsc.load_gather(
                                cur, [jnp.full((L,), ss, jnp.int32), vin])
                            plsc.store_scatter(
                                res, [jnp.full((L,), ss, jnp.int32), b], vals)
                        return cc

                    lax.fori_loop(boffs[kk] // L, boffs[kk + 1] // L,
                                  extract, 0)
                if TW:
                    kk = n_bkt - 1

                    def extract_tail(t, cc):
                        p = bq[pl.ds(t * L, L)]
                        v = p & ((1 << B_SHIFT) - 1)
                        b = lax.shift_right_logical(p, B_SHIFT)
                        vin = v - VF
                        for ss in range(8):
                            gidx = (dt * 8 + ss) * TW + vin
                            vals = plsc.load_gather(tailv, [gidx])
                            plsc.store_scatter(
                                res, [jnp.full((L,), ss, jnp.int32), b], vals)
                        return cc

                    lax.fori_loop(boffs[kk] // L, boffs[kk + 1] // L,
                                  extract_tail, 0)
                pltpu.sync_copy(
                    res.at[:, pl.ds(0, B)],
                    out_hbm.at[1 + i, pl.ds(dt * 8, 8), :])
                return carry

            lax.fori_loop(dlo, dhi, dt_body, 0)

        base = (wid * n_units) // NW
        cnt = ((wid + 1) * n_units) // NW - base
        iA = base // DT
        dloA = lax.rem(base, DT)
        dhiA = jnp.minimum(jnp.int32(DT), dloA + cnt)

        @pl.when(dhiA > dloA)
        def phase_a():
            stream_phase(iA, dloA, dhiA)

        iB = iA + 1
        dhiB = jnp.minimum((base + cnt) - iB * DT, jnp.int32(DT))

        @pl.when((dhiB > 0) & (iB < N_CAT))
        def phase_b():
            stream_phase(iB, jnp.int32(0), dhiB)

        pltpu.sync_copy(w_hbm, wv)
        pltpu.sync_copy(bias_hbm, bv)
        pltpu.sync_copy(cls_hbm, clsv)

        def aux_body(u, carry):
            aux = wid + u * NW

            @pl.when(aux < DT)
            def cls_block():
                dt = aux
                for ss in range(8):
                    cvec = jnp.full((L,), ss, jnp.int32) + dt * 8
                    cval = plsc.load_gather(clsv, [cvec])

                    def fill(g, cc, ss=ss, cval=cval):
                        res[ss, pl.ds(g * L, L)] = cval
                        return cc

                    lax.fori_loop(0, n_groups, fill, 0)
                pltpu.sync_copy(
                    res.at[:, pl.ds(0, B)],
                    out_hbm.at[0, pl.ds(dt * 8, 8), :])

            @pl.when((aux >= DT) & (aux < n_aux))
            def num_block():
                j = (aux - DT) // DT
                dt = lax.rem(aux - DT, DT)
                pltpu.sync_copy(num_hbm.at[j], nrow)
                for ss in range(8):
                    widx = jnp.full((L,), ss, jnp.int32) + (j * 64 + dt * 8)
                    wval = plsc.load_gather(wv, [widx])
                    bval = plsc.load_gather(bv, [widx])

                    def fill(g, cc, ss=ss, wval=wval, bval=bval):
                        x = nrow[pl.ds(g * L, L)]
                        res[ss, pl.ds(g * L, L)] = x * wval + bval
                        return cc

                    lax.fori_loop(0, n_groups, fill, 0)
                pltpu.sync_copy(
                    res.at[:, pl.ds(0, B)],
                    out_hbm.at[1 + N_CAT + j, pl.ds(dt * 8, 8), :])

            return carry

        lax.fori_loop(0, (n_aux + NW - 1) // NW, aux_body, 0)

    return k(cat_t, num_t, emb_t, tail_t, w_flat, b_flat, cls_flat)


def kernel(cat, num, emb_cat, w_num, b_num, cls):
    B = cat.shape[0]
    n_cat, vocab, D = emb_cat.shape
    VF = vocab // 128 * 128
    tail_t = jnp.transpose(emb_cat[:, VF:, :], (0, 2, 1)).reshape(-1)
    out_k = _tokenize(
        cat.T,
        num.T,
        jnp.transpose(emb_cat, (0, 2, 1)),
        tail_t,
        w_num.reshape(-1),
        b_num.reshape(-1),
        cls.reshape(-1),
        B, D, vocab,
    )
    return jnp.transpose(out_k, (2, 0, 1))


# speedup vs baseline: 3.7598x; 1.0600x over previous
import functools
import jax
import jax.numpy as jnp
from jax import lax
from jax.experimental import pallas as pl
from jax.experimental.pallas import tpu as pltpu
from jax.experimental.pallas import tpu_sc as plsc

N_CAT = 26
N_NUM = 13
N_TOK = 1 + N_CAT + N_NUM
L = 16
CHV = 8192
CH_SHIFT = 13
B_SHIFT = 17


def _sc_info():
    try:
        info = plsc.get_sparse_core_info()
        return info.num_cores, info.num_subcores
    except Exception:
        return 2, 16


@functools.partial(jax.jit, static_argnums=(7, 8, 9))
def _tokenize(cat_t, num_t, emb_t, tail_t, w_flat, b_flat, cls_flat, B, D, V):
    NC, NS = _sc_info()
    NW = NC * NS
    VF = V // 128 * 128
    TW = V - VF
    n_full = VF // CHV
    v_rem = VF - n_full * CHV
    chunks = [(kk * CHV, CHV) for kk in range(n_full)]
    if v_rem:
        chunks.append((n_full * CHV, v_rem))
    n_chunks = len(chunks)
    n_bkt = n_chunks + (1 if TW else 0)
    bq_cap = B + n_bkt * 2 * L
    n_groups = B // L
    DT = D // 8
    n_units = N_CAT * DT
    n_aux = DT + N_NUM * DT

    mesh = plsc.VectorSubcoreMesh(core_axis_name="c", subcore_axis_name="s")

    @functools.partial(
        pl.kernel,
        out_type=jax.ShapeDtypeStruct((N_TOK, D, B), jnp.float32),
        mesh=mesh,
        scratch_types=[
            pltpu.VMEM((B,), jnp.int32),
            pltpu.VMEM((bq_cap,), jnp.int32),
            pltpu.VMEM((8, CHV), jnp.float32),
            pltpu.VMEM((8, B + 128), jnp.float32),
            pltpu.VMEM((max(D * TW, 1),), jnp.float32),
            pltpu.VMEM((B,), jnp.float32),
            pltpu.VMEM((N_NUM * 64,), jnp.float32),
            pltpu.VMEM((N_NUM * 64,), jnp.float32),
            pltpu.VMEM((64,), jnp.float32),
        ],
        compiler_params=pltpu.CompilerParams(
            use_tc_tiling_on_sc=True, needs_layout_passes=False),
    )
    def k(cat_hbm, num_hbm, emb_hbm, tail_hbm, w_hbm, bias_hbm, cls_hbm,
          out_hbm, qv, bq, cbuf, res, tailv, nrow, wv, bv, clsv):
        c = lax.axis_index("c")
        s_ax = lax.axis_index("s")
        wid = s_ax * NC + c
        lanes = lax.iota(jnp.int32, L)

        def stream_phase(i, dlo, dhi):
            pltpu.sync_copy(cat_hbm.at[i], qv)
            if TW:
                pltpu.sync_copy(tail_hbm.at[pl.ds(i * D * TW, D * TW)], tailv)

            boffs = []
            pos = jnp.int32(0)
            for kk in range(n_bkt):
                boffs.append(pos)
                if TW and kk == n_bkt - 1:
                    lo, hi = VF, V
                else:
                    lo = chunks[kk][0]
                    hi = lo + chunks[kk][1]

                def mask_of(v, lo=lo, hi=hi, kk=kk):
                    if TW and kk == n_bkt - 1:
                        return v >= lo
                    m = lax.shift_right_logical(v, CH_SHIFT) == kk
                    if hi == VF and TW:
                        m = m & (v < VF)
                    return m

                def scan(g, p):
                    g4 = g * 16
                    vs = [qv[pl.ds((g4 + r) * L, L)] for r in range(16)]
                    ms = [mask_of(v) for v in vs]
                    incs = [jnp.where(m, jnp.int32(1), jnp.int32(0))
                            for m in ms]
                    cums = [plsc.cumsum(inc) for inc in incs]
                    ts = [jnp.sum(inc) for inc in incs]
                    pk = [vs[r] | (((g4 + r) * L + lanes) << B_SHIFT)
                          for r in range(16)]
                    off = p
                    for r in range(16):
                        plsc.store_scatter(bq, [off + cums[r] - 1], pk[r],
                                           mask=ms[r])
                        off = off + ts[r]
                    return off

                pos = lax.fori_loop(0, n_groups // 16, scan, pos)
                npad = (-pos) & (2 * L - 1)
                dummy = jnp.int32(lo) | ((B + lanes) << B_SHIFT)
                plsc.store_scatter(bq, [pos + lanes], dummy,
                                   mask=lanes < npad)
                plsc.store_scatter(bq, [pos + L + lanes], dummy,
                                   mask=lanes + L < npad)
                pos = pos + npad
            boffs.append(pos)

            def dt_body(dt, carry):
                for kk, (vlo, vlen) in enumerate(chunks):
                    pltpu.sync_copy(
                        emb_hbm.at[i, pl.ds(dt * 8, 8), pl.ds(vlo, vlen)],
                        cbuf.at[:, pl.ds(0, vlen)],
                    )

                    def extract(t, cc, vlo=vlo, cur=cbuf):
                        pa = bq[pl.ds(t * 2 * L, L)]
                        pb = bq[pl.ds((t * 2 + 1) * L, L)]
                        va = pa & ((1 << B_SHIFT) - 1)
                        vb = pb & ((1 << B_SHIFT) - 1)
                        ba = lax.shift_right_logical(pa, B_SHIFT)
                        bb = lax.shift_right_logical(pb, B_SHIFT)
                        vina = va - vlo
                        vinb = vb - vlo
                        for ss in range(8):
                            row = jnp.full((L,), ss, jnp.int32)
                            valsa = plsc.load_gather(cur, [row, vina])
                            valsb = plsc.load_gather(cur, [row, vinb])
                            plsc.store_scatter(res, [row, ba], valsa)
                            plsc.store_scatter(res, [row, bb], valsb)
                        return cc

                    lax.fori_loop(boffs[kk] // (2 * L),
                                  boffs[kk + 1] // (2 * L), extract, 0)
                if TW:
                    kk = n_bkt - 1

                    def extract_tail(t, cc):
                        pa = bq[pl.ds(t * 2 * L, L)]
                        pb = bq[pl.ds((t * 2 + 1) * L, L)]
                        va = pa & ((1 << B_SHIFT) - 1)
                        vb = pb & ((1 << B_SHIFT) - 1)
                        ba = lax.shift_right_logical(pa, B_SHIFT)
                        bb = lax.shift_right_logical(pb, B_SHIFT)
                        vina = va - VF
                        vinb = vb - VF
                        for ss in range(8):
                            row = jnp.full((L,), ss, jnp.int32)
                            gbase = (dt * 8 + ss) * TW
                            valsa = plsc.load_gather(tailv, [gbase + vina])
                            valsb = plsc.load_gather(tailv, [gbase + vinb])
                            plsc.store_scatter(res, [row, ba], valsa)
                            plsc.store_scatter(res, [row, bb], valsb)
                        return cc

                    lax.fori_loop(boffs[kk] // (2 * L),
                                  boffs[kk + 1] // (2 * L), extract_tail, 0)
                pltpu.sync_copy(
                    res.at[:, pl.ds(0, B)],
                    out_hbm.at[1 + i, pl.ds(dt * 8, 8), :])
                return carry

            lax.fori_loop(dlo, dhi, dt_body, 0)

        base = (wid * n_units) // NW
        cnt = ((wid + 1) * n_units) // NW - base
        iA = base // DT
        dloA = lax.rem(base, DT)
        dhiA = jnp.minimum(jnp.int32(DT), dloA + cnt)

        @pl.when(dhiA > dloA)
        def phase_a():
            stream_phase(iA, dloA, dhiA)

        iB = iA + 1
        dhiB = jnp.minimum((base + cnt) - iB * DT, jnp.int32(DT))

        @pl.when((dhiB > 0) & (iB < N_CAT))
        def phase_b():
            stream_phase(iB, jnp.int32(0), dhiB)

        pltpu.sync_copy(w_hbm, wv)
        pltpu.sync_copy(bias_hbm, bv)
        pltpu.sync_copy(cls_hbm, clsv)

        def aux_body(u, carry):
            aux = wid + u * NW

            @pl.when(aux < DT)
            def cls_block():
                dt = aux
                for ss in range(8):
                    cvec = jnp.full((L,), ss, jnp.int32) + dt * 8
                    cval = plsc.load_gather(clsv, [cvec])

                    def fill(g, cc, ss=ss, cval=cval):
                        res[ss, pl.ds(g * L, L)] = cval
                        return cc

                    lax.fori_loop(0, n_groups, fill, 0)
                pltpu.sync_copy(
                    res.at[:, pl.ds(0, B)],
                    out_hbm.at[0, pl.ds(dt * 8, 8), :])

            @pl.when((aux >= DT) & (aux < n_aux))
            def num_block():
                j = (aux - DT) // DT
                dt = lax.rem(aux - DT, DT)
                pltpu.sync_copy(num_hbm.at[j], nrow)
                for ss in range(8):
                    widx = jnp.full((L,), ss, jnp.int32) + (j * 64 + dt * 8)
                    wval = plsc.load_gather(wv, [widx])
                    bval = plsc.load_gather(bv, [widx])

                    def fill(g, cc, ss=ss, wval=wval, bval=bval):
                        x = nrow[pl.ds(g * L, L)]
                        res[ss, pl.ds(g * L, L)] = x * wval + bval
                        return cc

                    lax.fori_loop(0, n_groups, fill, 0)
                pltpu.sync_copy(
                    res.at[:, pl.ds(0, B)],
                    out_hbm.at[1 + N_CAT + j, pl.ds(dt * 8, 8), :])

            return carry

        lax.fori_loop(0, (n_aux + NW - 1) // NW, aux_body, 0)

    return k(cat_t, num_t, emb_t, tail_t, w_flat, b_flat, cls_flat)


def kernel(cat, num, emb_cat, w_num, b_num, cls):
    B = cat.shape[0]
    n_cat, vocab, D = emb_cat.shape
    VF = vocab // 128 * 128
    tail_t = jnp.transpose(emb_cat[:, VF:, :], (0, 2, 1)).reshape(-1)
    out_k = _tokenize(
        cat.T,
        num.T,
        jnp.transpose(emb_cat, (0, 2, 1)),
        tail_t,
        w_num.reshape(-1),
        b_num.reshape(-1),
        cls.reshape(-1),
        B, D, vocab,
    )
    return jnp.transpose(out_k, (2, 0, 1))
